# baseline (speedup 1.0000x reference)
import functools

import jax
import jax.numpy as jnp
from jax import lax
from jax.experimental import pallas as pl
from jax.experimental.pallas import tpu as pltpu
from jax.experimental.pallas import tpu_sc as plsc

_TOPK = 16


def _tc_body(fv_ref, ctx_ref, wq_ref, wk_ref, wv_ref, idxg_ref, smo_ref,
             idx_ref, vals_ref):
    fv = fv_ref[0]
    ctx = ctx_ref[0]
    n, d = fv.shape
    m = ctx.shape[0]

    q = jnp.dot(fv, wq_ref[...], preferred_element_type=jnp.float32)
    kk = jnp.dot(ctx, wk_ref[...], preferred_element_type=jnp.float32)
    vv = jnp.dot(ctx, wv_ref[...], preferred_element_type=jnp.float32)
    kk = kk - jnp.max(kk, axis=0, keepdims=True)
    ek = jnp.exp(kk)
    kk = ek / jnp.sum(ek, axis=0, keepdims=True)
    lam = lax.dot_general(kk, vv, (((0,), (0,)), ((), ())),
                          preferred_element_type=jnp.float32)
    w = jnp.dot(q, lam, preferred_element_type=jnp.float32)

    iota_f = lax.broadcasted_iota(jnp.int32, (n, m), 1).astype(jnp.float32)
    for t in range(_TOPK):
        mx = jnp.max(w, axis=1, keepdims=True)
        am = jnp.min(jnp.where(w == mx, iota_f, float(m)), axis=1,
                     keepdims=True)
        idx_ref[:, t] = am[:, 0]
        vals_ref[:, t] = mx[:, 0]
        w = jnp.where(iota_f == am, -jnp.inf, w)

    vals = vals_ref[...]
    sm = jnp.exp(vals - jnp.max(vals, axis=1, keepdims=True))
    sm = sm / jnp.sum(sm, axis=1, keepdims=True)

    b = pl.program_id(0)
    idxg_ref[...] = idx_ref[...].astype(jnp.int32) + b * m
    smb = jnp.broadcast_to(sm[:, :, None], (n, _TOPK, 16))
    smo_ref[...] = smb.reshape(n * _TOPK, 16)


def _tc_stage(featureVec, contextVec, Wq, Wk, Wv):
    b, n, d = featureVec.shape
    m = contextVec.shape[1]
    return pl.pallas_call(
        _tc_body,
        grid=(b,),
        in_specs=[
            pl.BlockSpec((1, n, d), lambda i: (i, 0, 0)),
            pl.BlockSpec((1, m, d), lambda i: (i, 0, 0)),
            pl.BlockSpec((d, d), lambda i: (0, 0)),
            pl.BlockSpec((d, d), lambda i: (0, 0)),
            pl.BlockSpec((d, m), lambda i: (0, 0)),
        ],
        out_specs=[
            pl.BlockSpec((n, _TOPK), lambda i: (i, 0)),
            pl.BlockSpec((n * _TOPK, 16), lambda i: (i, 0)),
        ],
        out_shape=[
            jax.ShapeDtypeStruct((b * n, _TOPK), jnp.int32),
            jax.ShapeDtypeStruct((b * n * _TOPK, 16), jnp.float32),
        ],
        scratch_shapes=[
            pltpu.VMEM((n, _TOPK), jnp.float32),
            pltpu.VMEM((n, _TOPK), jnp.float32),
        ],
    )(featureVec, contextVec, Wq, Wk, Wv)


def _make_sc_stage(bn, d):
    info = plsc.get_sparse_core_info()
    nc, ns = info.num_cores, info.num_subcores
    nw = nc * ns
    rows = bn // nw
    rb = 8
    nb = rows // rb

    mesh = plsc.VectorSubcoreMesh(core_axis_name="c", subcore_axis_name="s")

    @functools.partial(
        pl.kernel, mesh=mesh,
        out_type=jax.ShapeDtypeStruct((bn * _TOPK, d), jnp.float32),
        scratch_types=[
            pltpu.VMEM((rows * _TOPK,), jnp.int32),
            pltpu.VMEM((rows * _TOPK, 16), jnp.float32),
            pltpu.VMEM((rb, d), jnp.float32),
            pltpu.VMEM((rb * _TOPK, d), jnp.float32),
            pltpu.SemaphoreType.DMA,
        ],
    )
    def sck(idx_hbm, smf_hbm, fv_hbm, ctx_hbm, out_hbm,
            idx_v, smf_v, fv_v, rows_v, sem):
        wid = lax.axis_index("s") * nc + lax.axis_index("c")
        base = wid * rows
        pltpu.sync_copy(idx_hbm.at[pl.ds(base * _TOPK, rows * _TOPK)], idx_v)
        pltpu.sync_copy(smf_hbm.at[pl.ds(base * _TOPK, rows * _TOPK)], smf_v)
        for blk in range(nb):
            rb0 = base + blk * rb
            pltpu.sync_copy(fv_hbm.at[pl.ds(rb0, rb)], fv_v)
            pltpu.async_copy(
                ctx_hbm.at[idx_v.at[pl.ds(blk * rb * _TOPK, rb * _TOPK)]],
                rows_v, sem).wait()

            def row_body(r, carry):
                for t in range(_TOPK):
                    o = r * _TOPK + t
                    smt = smf_v[blk * rb * _TOPK + o, :]
                    for c in range(d // 16):
                        cs = pl.ds(c * 16, 16)
                        rows_v[o, cs] = rows_v[o, cs] * fv_v[r, cs] * smt
                return carry

            lax.fori_loop(0, rb, row_body, 0)
            pltpu.sync_copy(rows_v, out_hbm.at[pl.ds(rb0 * _TOPK, rb * _TOPK)])

    return sck


@jax.jit
def kernel(featureVec, contextVec, Wq, Wk, Wv):
    b, n, d = featureVec.shape
    m = contextVec.shape[1]
    idxg, sm = _tc_stage(featureVec, contextVec, Wq, Wk, Wv)
    sck = _make_sc_stage(b * n, d)
    out = sck(idxg.reshape(-1), sm,
              featureVec.reshape(b * n, d), contextVec.reshape(b * m, d))
    return out.reshape(b, n * _TOPK, d)


# speedup vs baseline: 1.1860x; 1.1860x over previous
import functools

import jax
import jax.numpy as jnp
from jax import lax
from jax.experimental import pallas as pl
from jax.experimental.pallas import tpu as pltpu
from jax.experimental.pallas import tpu_sc as plsc

_TOPK = 16


def _tc_body(fv_ref, ctx_ref, wq_ref, wk_ref, wv_ref, idxg_ref, smo_ref,
             idx_ref, vals_ref):
    fv = fv_ref[0]
    ctx = ctx_ref[0]
    n, d = fv.shape
    m = ctx.shape[0]

    q = jnp.dot(fv, wq_ref[...], preferred_element_type=jnp.float32)
    kk = jnp.dot(ctx, wk_ref[...], preferred_element_type=jnp.float32)
    vv = jnp.dot(ctx, wv_ref[...], preferred_element_type=jnp.float32)
    kk = kk - jnp.max(kk, axis=0, keepdims=True)
    ek = jnp.exp(kk)
    kk = ek / jnp.sum(ek, axis=0, keepdims=True)
    lam = lax.dot_general(kk, vv, (((0,), (0,)), ((), ())),
                          preferred_element_type=jnp.float32)
    w = jnp.dot(q, lam, preferred_element_type=jnp.float32)

    iota_f = lax.broadcasted_iota(jnp.int32, (n, m), 1).astype(jnp.float32)
    for t in range(_TOPK):
        mx = jnp.max(w, axis=1, keepdims=True)
        am = jnp.min(jnp.where(w == mx, iota_f, float(m)), axis=1,
                     keepdims=True)
        idx_ref[:, t] = am[:, 0]
        vals_ref[:, t] = mx[:, 0]
        w = jnp.where(iota_f == am, -jnp.inf, w)

    vals = vals_ref[...]
    sm = jnp.exp(vals - jnp.max(vals, axis=1, keepdims=True))
    sm = sm / jnp.sum(sm, axis=1, keepdims=True)

    b = pl.program_id(0)
    idxg_ref[...] = idx_ref[...].astype(jnp.int32) + b * m
    smb = jnp.broadcast_to(sm[:, :, None], (n, _TOPK, 16))
    smo_ref[...] = smb.reshape(n * _TOPK, 16)


def _tc_stage(featureVec, contextVec, Wq, Wk, Wv):
    b, n, d = featureVec.shape
    m = contextVec.shape[1]
    return pl.pallas_call(
        _tc_body,
        grid=(b,),
        in_specs=[
            pl.BlockSpec((1, n, d), lambda i: (i, 0, 0)),
            pl.BlockSpec((1, m, d), lambda i: (i, 0, 0)),
            pl.BlockSpec((d, d), lambda i: (0, 0)),
            pl.BlockSpec((d, d), lambda i: (0, 0)),
            pl.BlockSpec((d, m), lambda i: (0, 0)),
        ],
        out_specs=[
            pl.BlockSpec((n, _TOPK), lambda i: (i, 0)),
            pl.BlockSpec((n * _TOPK, 16), lambda i: (i, 0)),
        ],
        out_shape=[
            jax.ShapeDtypeStruct((b * n, _TOPK), jnp.int32),
            jax.ShapeDtypeStruct((b * n * _TOPK, 16), jnp.float32),
        ],
        scratch_shapes=[
            pltpu.VMEM((n, _TOPK), jnp.float32),
            pltpu.VMEM((n, _TOPK), jnp.float32),
        ],
    )(featureVec, contextVec, Wq, Wk, Wv)


def _make_sc_stage(bn, d):
    info = plsc.get_sparse_core_info()
    nc, ns = info.num_cores, info.num_subcores
    nw = nc * ns
    rows = bn // nw
    rb = 2
    nb = rows // rb

    mesh = plsc.VectorSubcoreMesh(core_axis_name="c", subcore_axis_name="s")

    @functools.partial(
        pl.kernel, mesh=mesh,
        out_type=jax.ShapeDtypeStruct((bn * _TOPK, d), jnp.float32),
        scratch_types=[
            pltpu.VMEM((rows * _TOPK,), jnp.int32),
            pltpu.VMEM((rows * _TOPK, 16), jnp.float32),
            pltpu.VMEM((rows, d), jnp.float32),
            pltpu.VMEM((rb * _TOPK, d), jnp.float32),
            pltpu.VMEM((rb * _TOPK, d), jnp.float32),
            pltpu.VMEM((rb * _TOPK, d), jnp.float32),
            pltpu.VMEM((rb * _TOPK, d), jnp.float32),
            pltpu.SemaphoreType.DMA,
            pltpu.SemaphoreType.DMA,
            pltpu.SemaphoreType.DMA,
            pltpu.SemaphoreType.DMA,
        ],
    )
    def sck(idx_hbm, smf_hbm, fv_hbm, ctx_hbm, out_hbm,
            idx_v, smf_v, fv_v, rows_a, rows_b, ob_a, ob_b,
            gs_a, gs_b, os_a, os_b):
        wid = lax.axis_index("s") * nc + lax.axis_index("c")
        base = wid * rows
        gbufs = (rows_a, rows_b)
        obufs = (ob_a, ob_b)
        gsems = (gs_a, gs_b)
        osems = (os_a, os_b)
        pltpu.sync_copy(idx_hbm.at[pl.ds(base * _TOPK, rows * _TOPK)], idx_v)
        pltpu.sync_copy(smf_hbm.at[pl.ds(base * _TOPK, rows * _TOPK)], smf_v)
        pltpu.sync_copy(fv_hbm.at[pl.ds(base, rows)], fv_v)

        def gather_start(blk, p):
            return pltpu.async_copy(
                ctx_hbm.at[idx_v.at[pl.ds(blk * rb * _TOPK, rb * _TOPK)]],
                gbufs[p], gsems[p])

        def gather_wait(blk, p):
            pltpu.make_async_copy(
                ctx_hbm.at[idx_v.at[pl.ds(blk * rb * _TOPK, rb * _TOPK)]],
                gbufs[p], gsems[p]).wait()

        def out_start(blk, p):
            pltpu.async_copy(
                obufs[p], out_hbm.at[pl.ds((base + blk * rb) * _TOPK,
                                           rb * _TOPK)], osems[p])

        def out_wait(blk, p):
            pltpu.make_async_copy(
                obufs[p], out_hbm.at[pl.ds((base + blk * rb) * _TOPK,
                                           rb * _TOPK)], osems[p]).wait()

        gather_start(0, 0)
        gather_start(1, 1)

        def pair_body(g, carry):
            for q in (0, 1):
                blk = g * 2 + q
                gather_wait(blk, q)

                @pl.when(blk >= 2)
                def _():
                    out_wait(blk - 2, q)

                def row_body(r, carry2):
                    gr = blk * rb + r
                    for t in range(_TOPK):
                        o = r * _TOPK + t
                        smt = smf_v[gr * _TOPK + t, :]
                        for c in range(d // 16):
                            cs = pl.ds(c * 16, 16)
                            obufs[q][o, cs] = (gbufs[q][o, cs]
                                               * fv_v[gr, cs] * smt)
                    return carry2

                lax.fori_loop(0, rb, row_body, 0)

                @pl.when(blk + 2 < nb)
                def _():
                    gather_start(blk + 2, q)

                out_start(blk, q)
            return carry

        lax.fori_loop(0, nb // 2, pair_body, 0)
        out_wait(nb - 2, 0)
        out_wait(nb - 1, 1)

    return sck


@jax.jit
def kernel(featureVec, contextVec, Wq, Wk, Wv):
    b, n, d = featureVec.shape
    m = contextVec.shape[1]
    idxg, sm = _tc_stage(featureVec, contextVec, Wq, Wk, Wv)
    sck = _make_sc_stage(b * n, d)
    out = sck(idxg.reshape(-1), sm,
              featureVec.reshape(b * n, d), contextVec.reshape(b * m, d))
    return out.reshape(b, n * _TOPK, d)


# speedup vs baseline: 1.6541x; 1.3948x over previous
import functools

import jax
import jax.numpy as jnp
from jax import lax
from jax.experimental import pallas as pl
from jax.experimental.pallas import tpu as pltpu
from jax.experimental.pallas import tpu_sc as plsc

_TOPK = 16


def _tc_body(fv_ref, ctx_ref, wq_ref, wk_ref, wv_ref, idxg_ref, smo_ref,
             idx_ref, vals_ref):
    fv = fv_ref[0]
    ctx = ctx_ref[0]
    n, d = fv.shape
    m = ctx.shape[0]

    q = jnp.dot(fv, wq_ref[...], preferred_element_type=jnp.float32)
    kk = jnp.dot(ctx, wk_ref[...], preferred_element_type=jnp.float32)
    vv = jnp.dot(ctx, wv_ref[...], preferred_element_type=jnp.float32)
    kk = kk - jnp.max(kk, axis=0, keepdims=True)
    ek = jnp.exp(kk)
    kk = ek / jnp.sum(ek, axis=0, keepdims=True)
    lam = lax.dot_general(kk, vv, (((0,), (0,)), ((), ())),
                          preferred_element_type=jnp.float32)
    w = jnp.dot(q, lam, preferred_element_type=jnp.float32)

    iota_f = lax.broadcasted_iota(jnp.int32, (n, m), 1).astype(jnp.float32)
    for t in range(_TOPK):
        mx = jnp.max(w, axis=1, keepdims=True)
        am = jnp.min(jnp.where(w == mx, iota_f, float(m)), axis=1,
                     keepdims=True)
        idx_ref[:, t] = am[:, 0]
        vals_ref[:, t] = mx[:, 0]
        w = jnp.where(iota_f == am, -jnp.inf, w)

    vals = vals_ref[...]
    sm = jnp.exp(vals - jnp.max(vals, axis=1, keepdims=True))
    sm = sm / jnp.sum(sm, axis=1, keepdims=True)

    b = pl.program_id(0)
    idxg_ref[...] = idx_ref[...].astype(jnp.int32) + b * m
    smb = jnp.broadcast_to(sm[:, :, None], (n, _TOPK, 16))
    smo_ref[...] = smb.reshape(n * _TOPK, 16)


def _tc_stage(featureVec, contextVec, Wq, Wk, Wv):
    b, n, d = featureVec.shape
    m = contextVec.shape[1]
    return pl.pallas_call(
        _tc_body,
        grid=(b,),
        in_specs=[
            pl.BlockSpec((1, n, d), lambda i: (i, 0, 0)),
            pl.BlockSpec((1, m, d), lambda i: (i, 0, 0)),
            pl.BlockSpec((d, d), lambda i: (0, 0)),
            pl.BlockSpec((d, d), lambda i: (0, 0)),
            pl.BlockSpec((d, m), lambda i: (0, 0)),
        ],
        out_specs=[
            pl.BlockSpec((n, _TOPK), lambda i: (i, 0)),
            pl.BlockSpec((n * _TOPK, 16), lambda i: (i, 0)),
        ],
        out_shape=[
            jax.ShapeDtypeStruct((b * n, _TOPK), jnp.int32),
            jax.ShapeDtypeStruct((b * n * _TOPK, 16), jnp.float32),
        ],
        scratch_shapes=[
            pltpu.VMEM((n, _TOPK), jnp.float32),
            pltpu.VMEM((n, _TOPK), jnp.float32),
        ],
    )(featureVec, contextVec, Wq, Wk, Wv)


def _make_sc_stage(bn, d):
    info = plsc.get_sparse_core_info()
    nc, ns = info.num_cores, info.num_subcores
    nw = nc * ns
    rows = bn // nw
    rb = 2
    nb = rows // rb

    mesh = plsc.VectorSubcoreMesh(core_axis_name="c", subcore_axis_name="s")

    @functools.partial(
        pl.kernel, mesh=mesh,
        out_type=jax.ShapeDtypeStruct((bn * _TOPK, d), jnp.float32),
        scratch_types=[
            pltpu.VMEM((rows * _TOPK,), jnp.int32),
            pltpu.VMEM((rows * _TOPK, 16), jnp.float32),
            pltpu.VMEM((rows, d), jnp.float32),
            pltpu.VMEM((rb * _TOPK, d), jnp.float32),
            pltpu.VMEM((rb * _TOPK, d), jnp.float32),
            pltpu.VMEM((rb * _TOPK, d), jnp.float32),
            pltpu.VMEM((rb * _TOPK, d), jnp.float32),
            pltpu.SemaphoreType.DMA,
            pltpu.SemaphoreType.DMA,
            pltpu.SemaphoreType.DMA,
            pltpu.SemaphoreType.DMA,
        ],
    )
    def sck(idx_hbm, smf_hbm, fv_hbm, ctx_hbm, out_hbm,
            idx_v, smf_v, fv_v, rows_a, rows_b, ob_a, ob_b,
            gs_a, gs_b, os_a, os_b):
        wid = lax.axis_index("s") * nc + lax.axis_index("c")
        base = wid * rows
        gbufs = (rows_a, rows_b)
        obufs = (ob_a, ob_b)
        gsems = (gs_a, gs_b)
        osems = (os_a, os_b)
        pltpu.sync_copy(idx_hbm.at[pl.ds(base * _TOPK, rows * _TOPK)], idx_v)
        pltpu.sync_copy(smf_hbm.at[pl.ds(base * _TOPK, rows * _TOPK)], smf_v)
        pltpu.sync_copy(fv_hbm.at[pl.ds(base, rows)], fv_v)

        def gather_start(blk, p):
            return pltpu.async_copy(
                ctx_hbm.at[idx_v.at[pl.ds(blk * rb * _TOPK, rb * _TOPK)]],
                gbufs[p], gsems[p])

        def gather_wait(blk, p):
            pltpu.make_async_copy(
                ctx_hbm.at[idx_v.at[pl.ds(blk * rb * _TOPK, rb * _TOPK)]],
                gbufs[p], gsems[p]).wait()

        def out_start(blk, p):
            pltpu.async_copy(
                obufs[p], out_hbm.at[pl.ds((base + blk * rb) * _TOPK,
                                           rb * _TOPK)], osems[p])

        def out_wait(blk, p):
            pltpu.make_async_copy(
                obufs[p], out_hbm.at[pl.ds((base + blk * rb) * _TOPK,
                                           rb * _TOPK)], osems[p]).wait()

        gather_start(0, 0)
        gather_start(1, 1)

        def pair_body(g, carry):
            for q in (0, 1):
                blk = g * 2 + q
                gather_wait(blk, q)

                @pl.when(blk >= 2)
                def _():
                    out_wait(blk - 2, q)

                for r in range(rb):
                    gr = blk * rb + r
                    sms = [smf_v[gr * _TOPK + t, :] for t in range(_TOPK)]
                    for c in range(d // 16):
                        cs = pl.ds(c * 16, 16)
                        fvc = fv_v[gr, cs]
                        for t in range(_TOPK):
                            o = r * _TOPK + t
                            obufs[q][o, cs] = gbufs[q][o, cs] * fvc * sms[t]

                @pl.when(blk + 2 < nb)
                def _():
                    gather_start(blk + 2, q)

                out_start(blk, q)
            return carry

        lax.fori_loop(0, nb // 2, pair_body, 0)
        out_wait(nb - 2, 0)
        out_wait(nb - 1, 1)

    return sck


@jax.jit
def kernel(featureVec, contextVec, Wq, Wk, Wv):
    b, n, d = featureVec.shape
    m = contextVec.shape[1]
    idxg, sm = _tc_stage(featureVec, contextVec, Wq, Wk, Wv)
    sck = _make_sc_stage(b * n, d)
    out = sck(idxg.reshape(-1), sm,
              featureVec.reshape(b * n, d), contextVec.reshape(b * m, d))
    return out.reshape(b, n * _TOPK, d)


# speedup vs baseline: 6.1831x; 3.7380x over previous
import functools

import jax
import jax.numpy as jnp
from jax import lax
from jax.experimental import pallas as pl
from jax.experimental.pallas import tpu as pltpu

_TOPK = 16
_NCHUNK = 4


def _body(fv_ref, ctx_ref, wq_ref, wk_ref, wv_ref, out_ref, idx_ref, vals_ref,
          obufs, sems):
    fv = fv_ref[0]
    ctx = ctx_ref[0]
    n, d = fv.shape
    m = ctx.shape[0]
    nb = pl.num_programs(0)

    q = jnp.dot(fv, wq_ref[...], preferred_element_type=jnp.float32)
    kk = jnp.dot(ctx, wk_ref[...], preferred_element_type=jnp.float32)
    vv = jnp.dot(ctx, wv_ref[...], preferred_element_type=jnp.float32)
    kk = kk - jnp.max(kk, axis=0, keepdims=True)
    ek = jnp.exp(kk)
    kk = ek / jnp.sum(ek, axis=0, keepdims=True)
    lam = lax.dot_general(kk, vv, (((0,), (0,)), ((), ())),
                          preferred_element_type=jnp.float32)
    w = jnp.dot(q, lam, preferred_element_type=jnp.float32)

    iota_f = lax.broadcasted_iota(jnp.int32, (n, m), 1).astype(jnp.float32)
    for t in range(_TOPK):
        mx = jnp.max(w, axis=1, keepdims=True)
        am = jnp.min(jnp.where(w == mx, iota_f, float(m)), axis=1,
                     keepdims=True)
        idx_ref[:, t] = am[:, 0]
        vals_ref[:, t] = mx[:, 0]
        w = jnp.where(iota_f == am, -jnp.inf, w)

    vals = vals_ref[...]
    sm = jnp.exp(vals - jnp.max(vals, axis=1, keepdims=True))
    sm = sm / jnp.sum(sm, axis=1, keepdims=True)
    sm_b = sm.astype(jnp.bfloat16)
    idx_b = idx_ref[...].astype(jnp.bfloat16)
    ctx_b = ctx.astype(jnp.bfloat16)

    b = pl.program_id(0)
    nh = n // _NCHUNK
    iota3 = lax.broadcasted_iota(jnp.int32, (nh, _TOPK, m), 2).astype(
        jnp.bfloat16)
    for h in range(_NCHUNK):
        lo, hi = h * nh, (h + 1) * nh
        sc = jnp.where(iota3 == idx_b[lo:hi, :][:, :, None],
                       sm_b[lo:hi, :][:, :, None],
                       jnp.bfloat16(0.0))
        g = jnp.dot(sc.reshape(nh * _TOPK, m), ctx_b,
                    preferred_element_type=jnp.float32)
        outh = (g.reshape(nh, _TOPK, d) * fv[lo:hi, :][:, None, :]
                ).reshape(nh * _TOPK, d)

        @pl.when(b > 0)
        def _(h=h):
            pltpu.make_async_copy(
                obufs[h],
                out_ref.at[b - 1, pl.ds(h * nh * _TOPK, nh * _TOPK), :],
                sems.at[h]).wait()

        obufs[h][...] = outh
        pltpu.make_async_copy(
            obufs[h],
            out_ref.at[b, pl.ds(h * nh * _TOPK, nh * _TOPK), :],
            sems.at[h]).start()

    @pl.when(b == nb - 1)
    def _():
        for h in range(_NCHUNK):
            pltpu.make_async_copy(
                obufs[h],
                out_ref.at[b, pl.ds(h * nh * _TOPK, nh * _TOPK), :],
                sems.at[h]).wait()


@jax.jit
def kernel(featureVec, contextVec, Wq, Wk, Wv):
    b, n, d = featureVec.shape
    m = contextVec.shape[1]
    nh = n // _NCHUNK

    def body(fv_ref, ctx_ref, wq_ref, wk_ref, wv_ref, out_ref,
             idx_ref, vals_ref, *rest):
        obufs = rest[:_NCHUNK]
        sems = rest[_NCHUNK]
        _body(fv_ref, ctx_ref, wq_ref, wk_ref, wv_ref, out_ref,
              idx_ref, vals_ref, obufs, sems)

    return pl.pallas_call(
        body,
        grid=(b,),
        in_specs=[
            pl.BlockSpec((1, n, d), lambda i: (i, 0, 0)),
            pl.BlockSpec((1, m, d), lambda i: (i, 0, 0)),
            pl.BlockSpec((d, d), lambda i: (0, 0)),
            pl.BlockSpec((d, d), lambda i: (0, 0)),
            pl.BlockSpec((d, m), lambda i: (0, 0)),
        ],
        out_specs=pl.BlockSpec(memory_space=pl.ANY),
        out_shape=jax.ShapeDtypeStruct((b, n * _TOPK, d), jnp.float32),
        scratch_shapes=[
            pltpu.VMEM((n, _TOPK), jnp.float32),
            pltpu.VMEM((n, _TOPK), jnp.float32),
        ] + [pltpu.VMEM((nh * _TOPK, d), jnp.float32)
             for _ in range(_NCHUNK)]
        + [pltpu.SemaphoreType.DMA((_NCHUNK,))],
    )(featureVec, contextVec, Wq, Wk, Wv)
